# Initial kernel scaffold; baseline (speedup 1.0000x reference)
import functools

import jax
import jax.numpy as jnp
from jax import lax
from jax.experimental import pallas as pl
from jax.experimental.pallas import tpu as pltpu
from jax.experimental.pallas import tpu_sc as plsc

_B = 16384
_V = 64
_D = 128
_NC = 2
_NS = 16
_NW = _NC * _NS
_BW = _B // _NW
_L = 16
_CHUNKS = _BW // _L


def _film_table_tc(t_ref, wt_ref, b_ref, o_ref):
    t = t_ref[...]
    wt = wt_ref[...]
    o_ref[...] = jnp.sum(wt[:, None, :] * t[None, :, :], axis=-1) + b_ref[...]


_film_table = pl.pallas_call(
    _film_table_tc,
    out_shape=jax.ShapeDtypeStruct((2, _V), jnp.float32),
)


_sc_mesh = plsc.VectorSubcoreMesh(core_axis_name="c", subcore_axis_name="s")


@functools.partial(
    pl.kernel,
    mesh=_sc_mesh,
    out_type=[
        jax.ShapeDtypeStruct((_B,), jnp.float32),
        jax.ShapeDtypeStruct((_B,), jnp.float32),
    ],
    scratch_types=[
        pltpu.VMEM((2, _V), jnp.float32),
        pltpu.VMEM((_BW,), jnp.int32),
        pltpu.VMEM((_BW,), jnp.float32),
        pltpu.VMEM((_BW,), jnp.float32),
    ],
)
def _gather_sc(ft_hbm, idx_hbm, g_hbm, bt_hbm, ft_v, idx_v, g_v, bt_v):
    wid = lax.axis_index("s") * _NC + lax.axis_index("c")
    base = wid * _BW
    pltpu.sync_copy(ft_hbm, ft_v)
    pltpu.sync_copy(idx_hbm.at[pl.ds(base, _BW)], idx_v)
    row0 = jnp.zeros((_L,), jnp.int32)
    row1 = row0 + 1
    for i in range(_CHUNKS):
        sl = pl.ds(i * _L, _L)
        idx = idx_v[sl]
        g_v[sl] = plsc.load_gather(ft_v, [row0, idx])
        bt_v[sl] = plsc.load_gather(ft_v, [row1, idx])
    pltpu.sync_copy(g_v, g_hbm.at[pl.ds(base, _BW)])
    pltpu.sync_copy(bt_v, bt_hbm.at[pl.ds(base, _BW)])


def kernel(effect_id, table, W, b):
    wt = W.T
    b2 = b.reshape(2, 1)
    ft = _film_table(table, wt, b2)
    idx = effect_id.reshape(_B)
    gamma, beta = _gather_sc(ft, idx)
    return gamma.reshape(_B, 1, 1), beta.reshape(_B, 1, 1)


# speedup vs baseline: 2.1057x; 2.1057x over previous
import functools

import jax
import jax.numpy as jnp
from jax import lax
from jax.experimental import pallas as pl
from jax.experimental.pallas import tpu as pltpu
from jax.experimental.pallas import tpu_sc as plsc

_B = 16384
_V = 64
_D = 128
_NC = 2
_NS = 16
_NW = _NC * _NS
_BW = _B // _NW
_L = 16
_CHUNKS = _BW // _L


def _film_table_tc(t_ref, wt_ref, b_ref, o_ref):
    t = t_ref[...]
    wt = wt_ref[...]
    o_ref[...] = jnp.sum(wt[:, None, :] * t[None, :, :], axis=-1) + b_ref[...]


_film_table = pl.pallas_call(
    _film_table_tc,
    out_shape=jax.ShapeDtypeStruct((2, _V), jnp.float32),
)


_sc_mesh = plsc.VectorSubcoreMesh(core_axis_name="c", subcore_axis_name="s")


@functools.partial(
    pl.kernel,
    mesh=_sc_mesh,
    out_type=[
        jax.ShapeDtypeStruct((_B,), jnp.float32),
        jax.ShapeDtypeStruct((_B,), jnp.float32),
    ],
    scratch_types=[
        pltpu.VMEM((_V,), jnp.float32),
        pltpu.VMEM((_V,), jnp.float32),
        pltpu.VMEM((_BW,), jnp.int32),
        pltpu.VMEM((_BW,), jnp.float32),
        pltpu.VMEM((_BW,), jnp.float32),
    ],
    compiler_params=pltpu.CompilerParams(needs_layout_passes=False),
)
def _gather_sc(ft_hbm, idx_hbm, g_hbm, bt_hbm, fg_v, fb_v, idx_v, g_v, bt_v):
    wid = lax.axis_index("s") * _NC + lax.axis_index("c")
    base = wid * _BW
    pltpu.sync_copy(ft_hbm.at[0], fg_v)
    pltpu.sync_copy(ft_hbm.at[1], fb_v)
    pltpu.sync_copy(idx_hbm.at[pl.ds(base, _BW)], idx_v)
    for i in range(_CHUNKS):
        sl = pl.ds(i * _L, _L)
        idx = idx_v[sl]
        g_v[sl] = plsc.load_gather(fg_v, [idx])
        bt_v[sl] = plsc.load_gather(fb_v, [idx])
    pltpu.sync_copy(g_v, g_hbm.at[pl.ds(base, _BW)])
    pltpu.sync_copy(bt_v, bt_hbm.at[pl.ds(base, _BW)])


def kernel(effect_id, table, W, b):
    wt = W.T
    b2 = b.reshape(2, 1)
    ft = _film_table(table, wt, b2)
    idx = effect_id.reshape(_B)
    gamma, beta = _gather_sc(ft, idx)
    return gamma.reshape(_B, 1, 1), beta.reshape(_B, 1, 1)


# speedup vs baseline: 2.2052x; 1.0473x over previous
import functools

import jax
import jax.numpy as jnp
from jax import lax
from jax.experimental import pallas as pl
from jax.experimental.pallas import tpu as pltpu
from jax.experimental.pallas import tpu_sc as plsc

_B = 16384
_V = 64
_D = 128
_NC = 2
_NS = 16
_NW = _NC * _NS
_BW = _B // _NW
_L = 16
_CHUNKS = _BW // _L


def _film_table_tc(t_ref, wt_ref, b_ref, o_ref):
    t = t_ref[...]
    wt = wt_ref[...]
    o_ref[...] = jnp.sum(wt[:, None, :] * t[None, :, :], axis=-1) + b_ref[...]


_film_table = pl.pallas_call(
    _film_table_tc,
    out_shape=jax.ShapeDtypeStruct((2, _V), jnp.float32),
)


_sc_mesh = plsc.VectorSubcoreMesh(core_axis_name="c", subcore_axis_name="s")


@functools.partial(
    pl.kernel,
    mesh=_sc_mesh,
    out_type=[
        jax.ShapeDtypeStruct((_B,), jnp.float32),
        jax.ShapeDtypeStruct((_B,), jnp.float32),
    ],
    scratch_types=[
        pltpu.VMEM((_V,), jnp.float32),
        pltpu.VMEM((_V,), jnp.float32),
        pltpu.VMEM((_BW,), jnp.int32),
        pltpu.VMEM((_BW,), jnp.float32),
        pltpu.VMEM((_BW,), jnp.float32),
        pltpu.SemaphoreType.DMA,
    ],
    compiler_params=pltpu.CompilerParams(needs_layout_passes=False),
)
def _gather_sc(ft_hbm, idx_hbm, g_hbm, bt_hbm, fg_v, fb_v, idx_v, g_v, bt_v, sem):
    wid = lax.axis_index("s") * _NC + lax.axis_index("c")
    base = wid * _BW
    c1 = pltpu.async_copy(ft_hbm.at[0], fg_v, sem)
    c2 = pltpu.async_copy(ft_hbm.at[1], fb_v, sem)
    c3 = pltpu.async_copy(idx_hbm.at[pl.ds(base, _BW)], idx_v, sem)
    c1.wait()
    c2.wait()
    c3.wait()
    for i in range(_CHUNKS):
        sl = pl.ds(i * _L, _L)
        idx = idx_v[sl]
        g_v[sl] = plsc.load_gather(fg_v, [idx])
        bt_v[sl] = plsc.load_gather(fb_v, [idx])
    c4 = pltpu.async_copy(g_v, g_hbm.at[pl.ds(base, _BW)], sem)
    c5 = pltpu.async_copy(bt_v, bt_hbm.at[pl.ds(base, _BW)], sem)
    c4.wait()
    c5.wait()


def kernel(effect_id, table, W, b):
    wt = W.T
    b2 = b.reshape(2, 1)
    ft = _film_table(table, wt, b2)
    idx = effect_id.reshape(_B)
    gamma, beta = _gather_sc(ft, idx)
    return gamma.reshape(_B, 1, 1), beta.reshape(_B, 1, 1)
